# Initial kernel scaffold; baseline (speedup 1.0000x reference)
import jax
import jax.numpy as jnp
from jax.experimental import pallas as pl


def kernel(feats, points1, points2, points3, neighbors1, neighbors2, neighbors3, subsampling1, subsampling2, upsampling1, upsampling2, params, kpoints):
    raise NotImplementedError("write your pallas kernel here")


# speedup vs baseline: 1.8244x; 1.8244x over previous
import jax
import jax.numpy as jnp
from jax import lax
from jax.experimental import pallas as pl
from jax.experimental.pallas import tpu as pltpu
from jax.experimental.pallas import tpu_sc as plsc

_NC, _NS = 2, 16
_NW = _NC * _NS
_CH = 128
_K = 15
_H = 32
_G = 32
_EPS = 1e-5


def _sc_gather(idx, tables):
    total, = idx.shape
    nt = len(tables)
    nc_full = total // _CH
    rem = total - nc_full * _CH
    nchunks = nc_full + (1 if rem else 0)
    nmax = -(-nchunks // _NW)

    outs = tuple(jax.ShapeDtypeStruct((total, t.shape[1]), jnp.float32)
                 for t in tables)
    scratch = [pltpu.VMEM((_CH,), jnp.int32)]
    scratch += [pltpu.VMEM((_CH, t.shape[1]), jnp.float32) for t in tables]
    scratch += [pltpu.SemaphoreType.DMA]
    mesh = plsc.VectorSubcoreMesh(core_axis_name="c", subcore_axis_name="s")

    def body(*refs):
        table_refs = refs[:nt]
        idx_ref = refs[nt]
        out_refs = refs[nt + 1:nt + 1 + nt]
        idx_v = refs[nt + 1 + nt]
        row_vs = refs[nt + 2 + nt:nt + 2 + 2 * nt]
        sem = refs[-1]
        wid = lax.axis_index("s") * _NC + lax.axis_index("c")

        def step(j, carry):
            ci = wid + j * _NW

            @pl.when(ci < nc_full)
            def _():
                base = ci * _CH
                pltpu.sync_copy(idx_ref.at[pl.ds(base, _CH)], idx_v)
                for t in range(nt):
                    pltpu.async_copy(table_refs[t].at[idx_v],
                                     row_vs[t], sem).wait()
                    pltpu.sync_copy(row_vs[t],
                                    out_refs[t].at[pl.ds(base, _CH)])

            if rem:
                @pl.when(ci == nc_full)
                def _():
                    base = nc_full * _CH
                    pltpu.sync_copy(idx_ref.at[pl.ds(base, rem)],
                                    idx_v.at[pl.ds(0, rem)])
                    for t in range(nt):
                        pltpu.async_copy(
                            table_refs[t].at[idx_v.at[pl.ds(0, rem)]],
                            row_vs[t].at[pl.ds(0, rem)], sem).wait()
                        pltpu.sync_copy(row_vs[t].at[pl.ds(0, rem)],
                                        out_refs[t].at[pl.ds(base, rem)])
            return carry

        lax.fori_loop(0, nmax, step, 0)

    fn = pl.kernel(body, out_type=outs, mesh=mesh, scratch_types=scratch,
                   compiler_params=pltpu.CompilerParams(
                       use_tc_tiling_on_sc=False))
    res = fn(*tables, idx)
    return res if isinstance(res, (tuple, list)) else (res,)


def _leaky(x):
    return jnp.where(x >= 0, x, 0.1 * x)


def _dot(a, b):
    return jnp.dot(a, b, preferred_element_type=jnp.float32,
                   precision=lax.Precision.HIGHEST)


def _dotd(a, b):
    return jnp.dot(a.astype(jnp.bfloat16), b.astype(jnp.bfloat16),
                   preferred_element_type=jnp.float32)


def _bf(x):
    return x.astype(jnp.bfloat16).astype(jnp.float32)


def _part_stats(y):
    c = y.shape[1]
    cg = c // _G
    s1 = jnp.sum(y, axis=0, keepdims=True)
    s2 = jnp.sum(y * y, axis=0, keepdims=True)
    rc = lax.broadcasted_iota(jnp.int32, (c, _G), 0)
    rg = lax.broadcasted_iota(jnp.int32, (c, _G), 1)
    et = (rc // cg == rg).astype(jnp.float32)
    gs1 = _dot(s1, et)
    gs2 = _dot(s2, et)
    return jnp.concatenate([gs1, gs2], axis=0).reshape(1, 2, _G)


def _stats_mi(p, c, n):
    cg = c // _G
    s = jnp.sum(p, axis=0)
    cnt = float(n * cg)
    m = s[0:1, :] / cnt
    v = s[1:2, :] / cnt - m * m
    inv = 1.0 / jnp.sqrt(v + _EPS)
    r2 = lax.broadcasted_iota(jnp.int32, (_G, c), 0)
    c2 = lax.broadcasted_iota(jnp.int32, (_G, c), 1)
    e = (c2 // cg == r2).astype(jnp.float32)
    mc = _dot(m, e)
    ic = _dot(inv, e)
    return mc, ic


def _gn_norm(y, mc, ic, g, bt):
    return (y - mc) * ic * g + bt


def _row(a):
    return a.reshape(1, -1)


def _full(shape):
    nd = len(shape)
    return pl.BlockSpec(shape, lambda i, _nd=nd: (0,) * _nd)


def _blk3(bq, c):
    return pl.BlockSpec((1, bq, c), lambda i: (i, 0, 0))


def _blk2(rows, c):
    return pl.BlockSpec((rows, c), lambda i: (i, 0))


def _kp_geometry(t2, q4, kpt_ref, mid, sigma):
    rows, d = t2.shape
    bq = q4.shape[0]
    h = rows // bq
    io = lax.broadcasted_iota(jnp.int32, (1, d), 1)
    kpt = kpt_ref[...]
    sq = None
    for j in range(3):
        ej = (io == mid + j).astype(jnp.float32)
        pj = jnp.sum(t2 * ej, axis=1, keepdims=True)
        qj = q4[:, j:j + 1]
        qjb = jnp.broadcast_to(qj.reshape(bq, 1, 1),
                               (bq, h, 1)).reshape(rows, 1)
        dj = (pj - qjb) - kpt[mid + j:mid + j + 1, :]
        sq = dj * dj if sq is None else sq + dj * dj
    return jnp.maximum(1.0 - jnp.sqrt(sq) * (1.0 / sigma), 0.0)


def _kp_first(t2d, qp3, kpt, w1p, sigma, bq):
    total, d = t2d.shape
    nq = total // _H
    nb = nq // bq
    cout = w1p.shape[1]

    def body(t_ref, qp_ref, kpt_ref, w1_ref, raw_ref, st_ref):
        t2 = t_ref[...]
        q4 = qp_ref[...].reshape(bq, 4)
        w = _kp_geometry(t2, q4, kpt_ref, 0, sigma)
        wsum = jnp.sum(_bf(w).reshape(bq, _H, 16), axis=1)
        raw = _dotd(wsum, w1_ref[...]) * (1.0 / _H)
        raw_ref[...] = raw.reshape(1, bq, cout)
        st_ref[...] = _part_stats(raw)

    return pl.pallas_call(
        body,
        grid=(nb,),
        in_specs=[_blk2(bq * _H, d), _blk3(bq, 4), _full(kpt.shape),
                  _full(w1p.shape)],
        out_specs=[_blk3(bq, cout), pl.BlockSpec((1, 2, _G),
                                                 lambda i: (i, 0, 0))],
        out_shape=[jax.ShapeDtypeStruct((nb, bq, cout), jnp.float32),
                   jax.ShapeDtypeStruct((nb, 2, _G), jnp.float32)],
    )(t2d, qp3, kpt, w1p)


def _kp_main(t2d, sf2d, qp3, kpt, wflat, sigma, bq, mid):
    total, d = t2d.shape
    nq = total // _H
    nb = nq // bq
    cout = wflat.shape[1]
    with_max = sf2d is not None

    def body(*refs):
        if with_max:
            t_ref, sf_ref, qp_ref, kpt_ref, wf_ref, raw_ref, st_ref, mx_ref = refs
        else:
            t_ref, qp_ref, kpt_ref, wf_ref, raw_ref, st_ref = refs
        t2 = t_ref[...]
        q4 = qp_ref[...].reshape(bq, 4)
        w = _kp_geometry(t2, q4, kpt_ref, mid, sigma)
        w3 = _bf(w).reshape(bq, _H, 16)
        io = lax.broadcasted_iota(jnp.int32, (1, d), 1)
        nm = (io < mid).astype(jnp.float32)
        rs = jnp.sum(t2 * nm, axis=1).reshape(bq, _H)
        cnt = jnp.sum((rs > 0.0).astype(jnp.float32), axis=1, keepdims=True)
        nnum = jnp.maximum(cnt, 1.0)
        nf3 = _bf(t_ref[...].reshape(bq, _H, d)[:, :, :mid])
        acc = jnp.zeros((bq, cout), jnp.float32)
        for k in range(_K):
            ak = jnp.sum(nf3 * w3[:, :, k:k + 1], axis=1)
            acc = acc + _dotd(ak, wf_ref[k * mid:(k + 1) * mid, :])
        raw = acc / nnum
        raw_ref[...] = raw.reshape(1, bq, cout)
        st_ref[...] = _part_stats(raw)
        if with_max:
            cin = sf_ref.shape[1]
            mx_ref[...] = jnp.max(sf_ref[...].reshape(bq, _H, cin),
                                  axis=1, keepdims=False).reshape(1, bq, cin)

    in_arrays = [t2d] + ([sf2d] if with_max else []) + [qp3, kpt, wflat]
    in_specs = [_blk2(bq * _H, d)]
    if with_max:
        cin = sf2d.shape[1]
        in_specs.append(_blk2(bq * _H, cin))
    in_specs += [_blk3(bq, 4), _full(kpt.shape), _full(wflat.shape)]
    out_shape = [jax.ShapeDtypeStruct((nb, bq, cout), jnp.float32),
                 jax.ShapeDtypeStruct((nb, 2, _G), jnp.float32)]
    out_specs = [_blk3(bq, cout),
                 pl.BlockSpec((1, 2, _G), lambda i: (i, 0, 0))]
    if with_max:
        out_shape.append(jax.ShapeDtypeStruct((nb, bq, cin), jnp.float32))
        out_specs.append(_blk3(bq, cin))

    res = pl.pallas_call(
        body,
        grid=(nb,),
        in_specs=in_specs,
        out_specs=out_specs,
        out_shape=out_shape,
    )(*in_arrays)
    if with_max:
        return res[0], res[1], res[2]
    return res[0], res[1], None


def _pass_mm(raw3, rawst, gng, gnb, u2w, u2b, sc3, scw, scb, nq):
    nb, bq, mid = raw3.shape
    cout = u2w.shape[1]
    has_sc = scw is not None

    def body(*refs):
        it = iter(refs)
        raw_ref, st_ref, gg, gb, u2w_r, u2b_r = (next(it) for _ in range(6))
        if has_sc:
            sc_ref, scw_r, scb_r = next(it), next(it), next(it)
        y2_ref, y2st_ref = next(it), next(it)
        if has_sc:
            s2_ref, s2st_ref = next(it), next(it)
        mc, ic = _stats_mi(st_ref[...], mid, nq)
        r = _leaky(_gn_norm(raw_ref[...].reshape(bq, mid), mc, ic,
                            gg[...], gb[...]))
        y2 = _dotd(r, u2w_r[...]) \
            + u2b_r[...]
        y2_ref[...] = y2.reshape(1, bq, cout)
        y2st_ref[...] = _part_stats(y2)
        if has_sc:
            cin = sc_ref.shape[2]
            s2 = _dotd(sc_ref[...].reshape(bq, cin), scw_r[...]) + scb_r[...]
            s2_ref[...] = s2.reshape(1, bq, cout)
            s2st_ref[...] = _part_stats(s2)

    args = [raw3, rawst, _row(gng), _row(gnb), u2w, _row(u2b)]
    in_specs = [_blk3(bq, mid), _full(rawst.shape), _full((1, mid)),
                _full((1, mid)), _full(u2w.shape), _full((1, cout))]
    if has_sc:
        cin = sc3.shape[2]
        args += [sc3, scw, _row(scb)]
        in_specs += [_blk3(bq, cin), _full(scw.shape), _full((1, cout))]
    out_shape = [jax.ShapeDtypeStruct((nb, bq, cout), jnp.float32),
                 jax.ShapeDtypeStruct((nb, 2, _G), jnp.float32)]
    out_specs = [_blk3(bq, cout),
                 pl.BlockSpec((1, 2, _G), lambda i: (i, 0, 0))]
    if has_sc:
        out_shape += [jax.ShapeDtypeStruct((nb, bq, cout), jnp.float32),
                      jax.ShapeDtypeStruct((nb, 2, _G), jnp.float32)]
        out_specs += [_blk3(bq, cout),
                      pl.BlockSpec((1, 2, _G), lambda i: (i, 0, 0))]

    res = pl.pallas_call(body, grid=(nb,), in_specs=in_specs,
                         out_specs=out_specs, out_shape=out_shape)(*args)
    if has_sc:
        return res[0], res[1], res[2], res[3]
    return res[0], res[1], None, None


def _pass_res(y23, y2st, u2g, u2bt, s3, s2st, scg, scbt, nu1w, nu1b, nq):
    nb, bq, cout = y23.shape
    gn_sc = s2st is not None
    has_next = nu1w is not None
    midn = nu1w.shape[1] if has_next else 0

    def body(*refs):
        it = iter(refs)
        y2_ref, y2st_ref, gg, gbt = (next(it) for _ in range(4))
        s_ref = next(it)
        if gn_sc:
            s2st_ref, sgg, sgbt = next(it), next(it), next(it)
        if has_next:
            nw_r, nb_r = next(it), next(it)
        x_ref = next(it)
        if has_next:
            u1y_ref, u1yst_ref = next(it), next(it)
        mc, ic = _stats_mi(y2st_ref[...], cout, nq)
        y = _gn_norm(y2_ref[...].reshape(bq, cout), mc, ic, gg[...], gbt[...])
        s = s_ref[...].reshape(bq, cout)
        if gn_sc:
            smc, sic = _stats_mi(s2st_ref[...], cout, nq)
            s = _gn_norm(s, smc, sic, sgg[...], sgbt[...])
        x = _leaky(y + s)
        x_ref[...] = x.reshape(1, bq, cout)
        if has_next:
            u1y = _dotd(x, nw_r[...]) + nb_r[...]
            u1y_ref[...] = u1y.reshape(1, bq, midn)
            u1yst_ref[...] = _part_stats(u1y)

    args = [y23, y2st, _row(u2g), _row(u2bt), s3]
    in_specs = [_blk3(bq, cout), _full(y2st.shape), _full((1, cout)),
                _full((1, cout)), _blk3(bq, cout)]
    if gn_sc:
        args += [s2st, _row(scg), _row(scbt)]
        in_specs += [_full(s2st.shape), _full((1, cout)), _full((1, cout))]
    if has_next:
        args += [nu1w, _row(nu1b)]
        in_specs += [_full(nu1w.shape), _full((1, midn))]
    out_shape = [jax.ShapeDtypeStruct((nb, bq, cout), jnp.float32)]
    out_specs = [_blk3(bq, cout)]
    if has_next:
        out_shape += [jax.ShapeDtypeStruct((nb, bq, midn), jnp.float32),
                      jax.ShapeDtypeStruct((nb, 2, _G), jnp.float32)]
        out_specs += [_blk3(bq, midn),
                      pl.BlockSpec((1, 2, _G), lambda i: (i, 0, 0))]

    res = pl.pallas_call(body, grid=(nb,), in_specs=in_specs,
                         out_specs=out_specs, out_shape=out_shape)(*args)
    if has_next:
        return res[0], res[1], res[2]
    return res[0] if isinstance(res, (list, tuple)) else res, None, None


def _pass_pack(u1y3, u1yst, g, bt, pts3, nq):
    nb, bq, midn = u1y3.shape

    def body(y_ref, st_ref, g_r, bt_r, p_ref, t_ref):
        mc, ic = _stats_mi(st_ref[...], midn, nq)
        u1x = _leaky(_gn_norm(y_ref[...].reshape(bq, midn), mc, ic,
                              g_r[...], bt_r[...]))
        t_ref[...] = jnp.concatenate(
            [u1x, p_ref[...].reshape(bq, 8)], axis=1).reshape(1, bq, midn + 8)

    return pl.pallas_call(
        body,
        grid=(nb,),
        in_specs=[_blk3(bq, midn), _full(u1yst.shape), _full((1, midn)),
                  _full((1, midn)), _blk3(bq, 8)],
        out_specs=_blk3(bq, midn + 8),
        out_shape=jax.ShapeDtypeStruct((nb, bq, midn + 8), jnp.float32),
    )(u1y3, u1yst, _row(g), _row(bt), pts3)


def _pass_first(raw3, rawst, gng, gnb, u1w, u1b, nq):
    nb, bq, c = raw3.shape
    midn = u1w.shape[1]

    def body(raw_ref, st_ref, gg, gb, uw, ub, x_ref, u1y_ref, u1yst_ref):
        mc, ic = _stats_mi(st_ref[...], c, nq)
        x = _leaky(_gn_norm(raw_ref[...].reshape(bq, c), mc, ic,
                            gg[...], gb[...]))
        x_ref[...] = x.reshape(1, bq, c)
        u1y = _dotd(x, uw[...]) \
            + ub[...]
        u1y_ref[...] = u1y.reshape(1, bq, midn)
        u1yst_ref[...] = _part_stats(u1y)

    return pl.pallas_call(
        body,
        grid=(nb,),
        in_specs=[_blk3(bq, c), _full(rawst.shape), _full((1, c)),
                  _full((1, c)), _full(u1w.shape), _full((1, midn))],
        out_specs=[_blk3(bq, c), _blk3(bq, midn),
                   pl.BlockSpec((1, 2, _G), lambda i: (i, 0, 0))],
        out_shape=[jax.ShapeDtypeStruct((nb, bq, c), jnp.float32),
                   jax.ShapeDtypeStruct((nb, bq, midn), jnp.float32),
                   jax.ShapeDtypeStruct((nb, 2, _G), jnp.float32)],
    )(raw3, rawst, _row(gng), _row(gnb), u1w, _row(u1b))


def _pass_dec2a(up3, sk3, w1, w2, b):
    nb, bq, cu = up3.shape
    cs = sk3.shape[2]
    cout = w1.shape[1]

    def body(u_ref, s_ref, w1_r, w2_r, b_r, y_ref, st_ref):
        y = (_dotd(u_ref[...].reshape(bq, cu), w1_r[...])
             + _dotd(s_ref[...].reshape(bq, cs), w2_r[...]) + b_r[...])
        y_ref[...] = y.reshape(1, bq, cout)
        st_ref[...] = _part_stats(y)

    return pl.pallas_call(
        body,
        grid=(nb,),
        in_specs=[_blk3(bq, cu), _blk3(bq, cs), _full(w1.shape),
                  _full(w2.shape), _full((1, cout))],
        out_specs=[_blk3(bq, cout),
                   pl.BlockSpec((1, 2, _G), lambda i: (i, 0, 0))],
        out_shape=[jax.ShapeDtypeStruct((nb, bq, cout), jnp.float32),
                   jax.ShapeDtypeStruct((nb, 2, _G), jnp.float32)],
    )(up3, sk3, w1, w2, _row(b))


def _pass_dec2b(y3, yst, g, bt, nq):
    nb, bq, c = y3.shape

    def body(y_ref, st_ref, g_r, bt_r, o_ref):
        mc, ic = _stats_mi(st_ref[...], c, nq)
        o_ref[...] = _leaky(_gn_norm(y_ref[...].reshape(bq, c), mc, ic,
                                     g_r[...], bt_r[...])).reshape(1, bq, c)

    return pl.pallas_call(
        body,
        grid=(nb,),
        in_specs=[_blk3(bq, c), _full(yst.shape), _full((1, c)),
                  _full((1, c))],
        out_specs=_blk3(bq, c),
        out_shape=jax.ShapeDtypeStruct((nb, bq, c), jnp.float32),
    )(y3, yst, _row(g), _row(bt))


def _pass_dec1(up3, sk3, w1, w2, b):
    nb, bq, cu = up3.shape
    cs = sk3.shape[2]
    cout = w1.shape[1]

    def body(u_ref, s_ref, w1_r, w2_r, b_r, o_ref):
        o_ref[...] = (_dotd(u_ref[...].reshape(bq, cu), w1_r[...])
                      + _dotd(s_ref[...].reshape(bq, cs), w2_r[...])
                      + b_r[...]).reshape(1, bq, cout)

    return pl.pallas_call(
        body,
        grid=(nb,),
        in_specs=[_blk3(bq, cu), _blk3(bq, cs), _full(w1.shape),
                  _full(w2.shape), _full((1, cout))],
        out_specs=_blk3(bq, cout),
        out_shape=jax.ShapeDtypeStruct((nb, bq, cout), jnp.float32),
    )(up3, sk3, w1, w2, _row(b))


def _pick_bq(n, target):
    bq = 1
    for d in range(1, min(n, target) + 1):
        if n % d == 0:
            bq = d
    return bq


def _pts8(p):
    n = p.shape[0]
    return jnp.concatenate([p, jnp.zeros((n, 5), jnp.float32)], axis=1)


def _qp3(p, bq):
    n = p.shape[0]
    q = jnp.concatenate([p, jnp.zeros((n, 1), jnp.float32)], axis=1)
    return q.reshape(n // bq, bq, 4)


def _kpt(kp, mid):
    z = jnp.zeros((mid + 8, 16), jnp.float32)
    return z.at[mid:mid + 3, :_K].set(kp.T)


def _wflat(w):
    k, ci, co = w.shape
    return w.reshape(k * ci, co)


def _w1pad(w):
    k, _, co = w.shape
    z = jnp.zeros((16, co), jnp.float32)
    return z.at[:k, :].set(w[:, 0, :])


def _to3(a, bq):
    n, c = a.shape
    return a.reshape(n // bq, bq, c)


def _to2(a):
    nb, bq, c = a.shape
    return a.reshape(nb * bq, c)


def _res_layer(idx_flat, u1x_table, sf_table, qp3, pts3_next, kp, p, p_next,
               sigma, bq, nq, sc_src3):
    mid = p['W'].shape[1]
    tables = [u1x_table] + ([sf_table] if sf_table is not None else [])
    g = _sc_gather(idx_flat, tables)
    t2d = g[0]
    sf2d = g[1] if sf_table is not None else None
    raw3, rawst, mx3 = _kp_main(t2d, sf2d, qp3, _kpt(kp, mid),
                                _wflat(p['W']), sigma, bq, mid)
    has_sc = 'sc' in p
    if sf_table is not None:
        s3 = mx3
    else:
        s3 = sc_src3
    if has_sc:
        y23, y2st, s23, s2st = _pass_mm(
            raw3, rawst, p['gn_g'], p['gn_b'], p['u2']['w'], p['u2']['b'],
            s3, p['sc']['w'], p['sc']['b'], nq)
        x3, u1y3, u1yst = _pass_res(
            y23, y2st, p['u2']['g'], p['u2']['bt'], s23, s2st,
            p['sc']['g'], p['sc']['bt'],
            p_next['u1']['w'] if p_next else None,
            p_next['u1']['b'] if p_next else None, nq)
    else:
        y23, y2st, _, _ = _pass_mm(
            raw3, rawst, p['gn_g'], p['gn_b'], p['u2']['w'], p['u2']['b'],
            None, None, None, nq)
        x3, u1y3, u1yst = _pass_res(
            y23, y2st, p['u2']['g'], p['u2']['bt'], s3, None, None, None,
            p_next['u1']['w'] if p_next else None,
            p_next['u1']['b'] if p_next else None, nq)
    if p_next is None:
        return x3, None
    t_next = _pass_pack(u1y3, u1yst, p_next['u1']['g'], p_next['u1']['bt'],
                        pts3_next, nq)
    return x3, _to2(t_next)


def kernel(feats, points1, points2, points3, neighbors1, neighbors2,
           neighbors3, subsampling1, subsampling2, upsampling1, upsampling2,
           params, kpoints):
    n1, n2, n3 = points1.shape[0], points2.shape[0], points3.shape[0]
    s1v, s2v, s3v = 0.1, 0.2, 0.4
    bq1, bq2, bq3 = _pick_bq(n1, 250), _pick_bq(n2, 250), _pick_bq(n3, 125)

    p1_8, p2_8, p3_8 = _pts8(points1), _pts8(points2), _pts8(points3)
    qp1, qp2, qp3 = _qp3(points1, bq1), _qp3(points2, bq2), _qp3(points3, bq3)
    p13, p23, p33 = _to3(p1_8, bq1), _to3(p2_8, bq2), _to3(p3_8, bq3)

    nb1 = neighbors1.reshape(-1).astype(jnp.int32)
    nb2 = neighbors2.reshape(-1).astype(jnp.int32)
    nb3 = neighbors3.reshape(-1).astype(jnp.int32)
    sb1 = subsampling1.reshape(-1).astype(jnp.int32)
    sb2 = subsampling2.reshape(-1).astype(jnp.int32)

    t0, = _sc_gather(nb1, [p1_8])
    raw3, rawst = _kp_first(t0, qp1, _kpt(kpoints['e1_1'], 0),
                            _w1pad(params['e1_1']['W']), s1v, bq1)
    x03, u1y3, u1yst = _pass_first(raw3, rawst, params['e1_1']['gn_g'],
                                   params['e1_1']['gn_b'],
                                   params['e1_2']['u1']['w'],
                                   params['e1_2']['u1']['b'], n1)
    t1 = _to2(_pass_pack(u1y3, u1yst, params['e1_2']['u1']['g'],
                         params['e1_2']['u1']['bt'], p13, n1))

    x13, t2 = _res_layer(nb1, t1, None, qp1, p13, kpoints['e1_2'],
                         params['e1_2'], params['e2_1'], s1v, bq1, n1, x03)
    x1 = _to2(x13)

    x23, t3 = _res_layer(sb1, t2, x1, qp2, p23, kpoints['e2_1'],
                         params['e2_1'], params['e2_2'], s1v, bq2, n2, None)

    x23b, t4 = _res_layer(nb2, t3, None, qp2, p23, kpoints['e2_2'],
                          params['e2_2'], params['e2_3'], s2v, bq2, n2, x23)

    x23c, t5 = _res_layer(nb2, t4, None, qp2, p23, kpoints['e2_3'],
                          params['e2_3'], params['e3_1'], s2v, bq2, n2, x23b)
    x2 = _to2(x23c)

    x33, t6 = _res_layer(sb2, t5, x2, qp3, p33, kpoints['e3_1'],
                         params['e3_1'], params['e3_2'], s2v, bq3, n3, None)

    x33b, t7 = _res_layer(nb3, t6, None, qp3, p33, kpoints['e3_2'],
                          params['e3_2'], params['e3_3'], s3v, bq3, n3, x33)

    x33c, _ = _res_layer(nb3, t7, None, qp3, None, kpoints['e3_3'],
                         params['e3_3'], None, s3v, bq3, n3, x33b)
    x3 = _to2(x33c)

    upg2, = _sc_gather(upsampling2[:, 0].astype(jnp.int32), [x3])
    bqd2 = _pick_bq(n2, 250)
    y3, yst = _pass_dec2a(_to3(upg2, bqd2), _to3(x2, bqd2),
                          params['dec2']['w'][:x3.shape[1]],
                          params['dec2']['w'][x3.shape[1]:],
                          params['dec2']['b'])
    l2 = _to2(_pass_dec2b(y3, yst, params['dec2']['g'], params['dec2']['bt'],
                          n2))

    upg1, = _sc_gather(upsampling1[:, 0].astype(jnp.int32), [l2])
    bqd1 = _pick_bq(n1, 500)
    l1 = _to2(_pass_dec1(_to3(upg1, bqd1), _to3(x1, bqd1),
                         params['dec1']['w'][:l2.shape[1]],
                         params['dec1']['w'][l2.shape[1]:],
                         params['dec1']['b']))
    return (l1, l2, x3)
